# baseline (speedup 1.0000x reference)
import jax
import jax.numpy as jnp
from jax.experimental import pallas as pl
from jax.experimental.pallas import tpu as pltpu

_BM = 256
_NBUF = 4
_SUB = 2
_OCHUNK = 8


def _body(a_hbm, e_hbm, o_hbm, b0, b1, b2, b3, ebuf, ebf, obuf, asem, esem, osem):
    M, K = a_hbm.shape
    nsteps = M // _BM
    bufs = [b0, b1, b2, b3]

    ecopy = pltpu.make_async_copy(e_hbm, ebuf, esem)
    ecopy.start()

    def a_copy(i):
        slot = i % _NBUF
        return pltpu.make_async_copy(
            a_hbm.at[pl.ds(i * _BM, _BM)], bufs[slot], asem.at[slot]
        )

    def o_copy(c):
        rows = _OCHUNK * _BM
        return pltpu.make_async_copy(
            obuf.at[pl.ds(c * rows, rows)],
            o_hbm.at[pl.ds(c * rows, rows)],
            osem,
        )

    for i in range(_NBUF - 1):
        a_copy(i).start()

    ecopy.wait()
    ebf[...] = ebuf[...].astype(jnp.bfloat16)

    h = _BM // _SUB
    for i in range(nsteps):
        a_copy(i).wait()
        nxt = i + _NBUF - 1
        if nxt < nsteps:
            a_copy(nxt).start()
        for s in range(_SUB):
            obuf[pl.ds(i * _BM + s * h, h)] = jax.lax.dot_general(
                bufs[i % _NBUF][pl.ds(s * h, h)], ebf[...],
                dimension_numbers=(((1,), (0,)), ((), ())),
                preferred_element_type=jnp.float32,
            )
        if (i + 1) % _OCHUNK == 0:
            o_copy(i // _OCHUNK).start()

    for c in range(nsteps // _OCHUNK):
        o_copy(c).wait()


def kernel(matrix_parents, Epsilon):
    M, K = matrix_parents.shape
    _, N = Epsilon.shape
    return pl.pallas_call(
        _body,
        in_specs=[
            pl.BlockSpec(memory_space=pl.ANY),
            pl.BlockSpec(memory_space=pl.ANY),
        ],
        out_specs=pl.BlockSpec(memory_space=pl.ANY),
        out_shape=jax.ShapeDtypeStruct((M, N), jnp.float32),
        scratch_shapes=[
            pltpu.VMEM((_BM, K), jnp.float32),
            pltpu.VMEM((_BM, K), jnp.float32),
            pltpu.VMEM((_BM, K), jnp.float32),
            pltpu.VMEM((_BM, K), jnp.float32),
            pltpu.VMEM((K, N), jnp.float32),
            pltpu.VMEM((K, N), jnp.bfloat16),
            pltpu.VMEM((M, N), jnp.float32),
            pltpu.SemaphoreType.DMA((_NBUF,)),
            pltpu.SemaphoreType.DMA,
            pltpu.SemaphoreType.DMA,
        ],
    )(matrix_parents, Epsilon)


# speedup vs baseline: 1.0289x; 1.0289x over previous
import jax
import jax.numpy as jnp
from jax.experimental import pallas as pl
from jax.experimental.pallas import tpu as pltpu

_BM = 256
_SUB = 2


def _body(a_ref, e_ref, o_ref):
    e_bf = e_ref[...].astype(jnp.bfloat16)
    h = _BM // _SUB
    for s in range(_SUB):
        o_ref[pl.ds(s * h, h)] = jax.lax.dot_general(
            a_ref[pl.ds(s * h, h)], e_bf,
            dimension_numbers=(((1,), (0,)), ((), ())),
            preferred_element_type=jnp.float32,
        )


def kernel(matrix_parents, Epsilon):
    M, K = matrix_parents.shape
    _, N = Epsilon.shape
    return pl.pallas_call(
        _body,
        grid=(M // _BM,),
        in_specs=[
            pl.BlockSpec((_BM, K), lambda i: (i, 0)),
            pl.BlockSpec((K, N), lambda i: (0, 0)),
        ],
        out_specs=pl.BlockSpec((_BM, N), lambda i: (i, 0)),
        out_shape=jax.ShapeDtypeStruct((M, N), jnp.float32),
        compiler_params=pltpu.CompilerParams(
            dimension_semantics=("arbitrary",),
            disable_bounds_checks=True,
        ),
    )(matrix_parents, Epsilon)
